# baseline (device time: 64060 ns/iter reference)
import os

import jax
import jax.numpy as jnp
from jax import lax
from jax.experimental import pallas as pl
from jax.experimental.pallas import tpu as pltpu

KMODE = os.environ.get("KMODE", "full")
KSCOPE = os.environ.get("KSCOPE", "0") == "1"

import contextlib


def _scope(name):
    return jax.named_scope(name) if KSCOPE else contextlib.nullcontext()

N_DEV = 16
B, SQ, DM = 2, 512, 768
DH = 64
BLK = 64
ROWS = B * SQ
SC = SQ // N_DEV


def kernel(x, Wq, K_ext, V_ext, Wo):
    H = K_ext.shape[2]
    HD = H * DH

    idx = lax.axis_index("i")
    x2 = x.reshape(ROWS, DM)
    k2 = K_ext.reshape(ROWS, HD)
    v2 = V_ext.reshape(ROWS, HD)
    wq_s = lax.dynamic_slice(Wq, (0, idx * HD), (DM, HD)).astype(jnp.bfloat16)
    wo_s = lax.dynamic_slice(Wo, (idx * HD, 0), (HD, DM)).astype(jnp.bfloat16)

    def body(x_ref, wq_ref, k_ref, v_ref, wo_ref, o_ref,
             xb_ref, kb_ref, vb_ref, q_ref, ctx_ref, part_ref,
             redA_ref, redB_ref, recvA, recvB, agbuf,
             rsA_s, rsA_r, rsB_s, rsB_r, agA_s, agA_r, agB_s, agB_r):
        me = lax.axis_index("i")

        with _scope("ph_prep"):
            xb_ref[...] = x_ref[...].astype(jnp.bfloat16)
            kb_ref[...] = k_ref[...].astype(jnp.bfloat16)
            vb_ref[...] = v_ref[...].astype(jnp.bfloat16)

            q_ref[...] = jnp.dot(
                xb_ref[...], wq_ref[...], preferred_element_type=jnp.float32
            ).astype(jnp.bfloat16)

        qb = lax.broadcasted_iota(jnp.int32, (SQ, SQ), 0) // BLK
        kb = lax.broadcasted_iota(jnp.int32, (SQ, SQ), 1) // BLK
        mask = (qb == kb) | (kb == 0) | ((qb + kb) % 3 == 0)
        bias = jnp.where(mask, 0.0, -1e9).astype(jnp.float32)

        def attention(b):
            rows = slice(b * SQ, (b + 1) * SQ)
            for h in range(H):
                if KMODE == "noattn":
                    break
                cols = slice(h * DH, (h + 1) * DH)
                s = lax.dot_general(
                    q_ref[rows, cols], kb_ref[rows, cols],
                    (((1,), (1,)), ((), ())),
                    preferred_element_type=jnp.float32,
                )
                e = jnp.exp(s * 0.125 + bias)
                rsum = jnp.sum(e, axis=1, keepdims=True)
                ctx = jnp.dot(e.astype(jnp.bfloat16), vb_ref[rows, cols],
                              preferred_element_type=jnp.float32)
                ctx = ctx * (1.0 / rsum)
                ctx_ref[rows, cols] = ctx.astype(jnp.bfloat16)

        def outproj(base, recv, ssem, rsem):
            descs = []
            for o in range(N_DEV):
                j = (me + o) % N_DEV
                rows = pl.ds(base + j * SC, SC)
                pc = jnp.dot(ctx_ref[rows, :], wo_ref[...],
                             preferred_element_type=jnp.float32
                             ).astype(jnp.bfloat16)
                part_ref[rows, :] = pc
                if o == 0:
                    recv[me, :, :] = pc
                else:
                    d = pltpu.make_async_remote_copy(
                        src_ref=part_ref.at[rows, :],
                        dst_ref=recv.at[me],
                        send_sem=ssem.at[o - 1],
                        recv_sem=rsem.at[o - 1],
                        device_id=(j,), device_id_type=pl.DeviceIdType.MESH)
                    d.start()
                    descs.append(d)
            return descs

        def reduce_chunk(recv, descs, red_ref):
            acc = recv[me].astype(jnp.float32)
            for o in range(1, N_DEV):
                descs[o - 1].wait_recv()
                acc = acc + recv[(me - o) % N_DEV].astype(jnp.float32)
            red_ref[...] = acc.astype(jnp.bfloat16)
            return acc

        def ag_push(base, red_ref, acc, ssem, rsem):
            myrows = pl.ds(base + me * SC, SC)
            agbuf[myrows, :] = red_ref[...]
            o_ref[myrows, :] = acc
            descs = []
            for o in range(1, N_DEV):
                j = (me + o) % N_DEV
                d = pltpu.make_async_remote_copy(
                    src_ref=red_ref,
                    dst_ref=agbuf.at[myrows, :],
                    send_sem=ssem.at[o - 1],
                    recv_sem=rsem.at[o - 1],
                    device_id=(j,), device_id_type=pl.DeviceIdType.MESH)
                d.start()
                descs.append(d)
            return descs

        def ag_collect(base, descs):
            for o in range(1, N_DEV):
                descs[o - 1].wait_recv()
                rows = pl.ds(base + ((me - o) % N_DEV) * SC, SC)
                o_ref[rows, :] = agbuf[rows, :].astype(jnp.float32)

        with _scope("ph_attn0"):
            attention(0)

        if KMODE == "nocomm":
            for b in range(1, B):
                attention(b)
            for o in range(N_DEV):
                j = (me + o) % N_DEV
                for base in (0, SQ):
                    rows = pl.ds(base + j * SC, SC)
                    o_ref[rows, :] = jnp.dot(
                        ctx_ref[rows, :], wo_ref[...],
                        preferred_element_type=jnp.float32)
            return

        if KMODE == "expbar":
            bar = pltpu.get_barrier_semaphore()
            for j in range(N_DEV):
                pl.semaphore_signal(bar, inc=1, device_id=(j,),
                                    device_id_type=pl.DeviceIdType.MESH)
            pl.semaphore_wait(bar, N_DEV)

        if KMODE == "justbar":
            for b in range(1, B):
                attention(b)
            for o in range(N_DEV):
                j = (me + o) % N_DEV
                for base in (0, SQ):
                    rows = pl.ds(base + j * SC, SC)
                    o_ref[rows, :] = jnp.dot(
                        ctx_ref[rows, :], wo_ref[...],
                        preferred_element_type=jnp.float32)
            return

        with _scope("ph_outprojA"):
            rsA = outproj(0, recvA, rsA_s, rsA_r)
        with _scope("ph_attn1"):
            attention(1)
        with _scope("ph_outprojB"):
            rsB = outproj(SQ, recvB, rsB_s, rsB_r)
        with _scope("ph_reduceA"):
            accA = reduce_chunk(recvA, rsA, redA_ref)

        if KMODE == "noag":
            accB = reduce_chunk(recvB, rsB, redB_ref)
            o_ref[pl.ds(me * SC, SC), :] = accA
            o_ref[pl.ds(SQ + me * SC, SC), :] = accB
            for d in rsA + rsB:
                d.wait_send()
            return

        with _scope("ph_agA"):
            agA = ag_push(0, redA_ref, accA, agA_s, agA_r)
        with _scope("ph_reduceB"):
            accB = reduce_chunk(recvB, rsB, redB_ref)
        with _scope("ph_agB"):
            agB = ag_push(SQ, redB_ref, accB, agB_s, agB_r)

        with _scope("ph_drain_rs"):
            for d in rsA + rsB:
                d.wait_send()
        with _scope("ph_collectA"):
            ag_collect(0, agA)
        with _scope("ph_collectB"):
            ag_collect(SQ, agB)
        with _scope("ph_drain_ag"):
            for d in agA + agB:
                d.wait_send()

    out = pl.pallas_call(
        body,
        out_shape=jax.ShapeDtypeStruct((ROWS, DM), jnp.float32),
        in_specs=[pl.BlockSpec(memory_space=pltpu.VMEM)] * 5,
        out_specs=pl.BlockSpec(memory_space=pltpu.VMEM),
        scratch_shapes=[
            pltpu.VMEM((ROWS, DM), jnp.bfloat16),
            pltpu.VMEM((ROWS, HD), jnp.bfloat16),
            pltpu.VMEM((ROWS, HD), jnp.bfloat16),
            pltpu.VMEM((ROWS, HD), jnp.bfloat16),
            pltpu.VMEM((ROWS, HD), jnp.bfloat16),
            pltpu.VMEM((ROWS, DM), jnp.bfloat16),
            pltpu.VMEM((SC, DM), jnp.bfloat16),
            pltpu.VMEM((SC, DM), jnp.bfloat16),
            pltpu.VMEM((N_DEV, SC, DM), jnp.bfloat16),
            pltpu.VMEM((N_DEV, SC, DM), jnp.bfloat16),
            pltpu.VMEM((ROWS, DM), jnp.bfloat16),
            pltpu.SemaphoreType.DMA((N_DEV - 1,)),
            pltpu.SemaphoreType.DMA((N_DEV - 1,)),
            pltpu.SemaphoreType.DMA((N_DEV - 1,)),
            pltpu.SemaphoreType.DMA((N_DEV - 1,)),
            pltpu.SemaphoreType.DMA((N_DEV - 1,)),
            pltpu.SemaphoreType.DMA((N_DEV - 1,)),
            pltpu.SemaphoreType.DMA((N_DEV - 1,)),
            pltpu.SemaphoreType.DMA((N_DEV - 1,)),
        ],
        compiler_params=pltpu.CompilerParams(
            collective_id=0 if KMODE == "expbar" else None),
    )(x2, wq_s, k2, v2, wo_s)

    return out.reshape(B, SQ, DM)


# device time: 57564 ns/iter; 1.1128x vs baseline; 1.1128x over previous
import os

import jax
import jax.numpy as jnp
from jax import lax
from jax.experimental import pallas as pl
from jax.experimental.pallas import tpu as pltpu

KMODE = os.environ.get("KMODE", "full")
KSCOPE = os.environ.get("KSCOPE", "0") == "1"

import contextlib


def _scope(name):
    return jax.named_scope(name) if KSCOPE else contextlib.nullcontext()

N_DEV = 16
B, SQ, DM = 2, 512, 768
DH = 64
BLK = 64
ROWS = B * SQ
SC = SQ // N_DEV


def kernel(x, Wq, K_ext, V_ext, Wo):
    H = K_ext.shape[2]
    HD = H * DH

    idx = lax.axis_index("i")
    x2 = x.reshape(ROWS, DM)
    k2 = K_ext.reshape(ROWS, HD)
    v2 = V_ext.reshape(ROWS, HD)
    wq_s = lax.dynamic_slice(Wq, (0, idx * HD), (DM, HD)).astype(jnp.bfloat16)
    wo_s = lax.dynamic_slice(Wo, (idx * HD, 0), (HD, DM)).astype(jnp.bfloat16)

    def body(x_ref, wq_ref, k_ref, v_ref, wo_ref, o_ref,
             xb_ref, kb_ref, vb_ref, q_ref, ctx_ref, part_ref,
             redA_ref, redB_ref, recvA, recvB, agbuf,
             rsA_s, rsA_r, rsB_s, rsB_r, agA_s, agA_r, agB_s, agB_r):
        me = lax.axis_index("i")

        if KMODE != "nocomm":
            bar = pltpu.get_barrier_semaphore()
            for j in range(N_DEV):
                pl.semaphore_signal(bar, inc=1, device_id=(j,),
                                    device_id_type=pl.DeviceIdType.MESH)

        with _scope("ph_prep"):
            xb_ref[...] = x_ref[...].astype(jnp.bfloat16)
            kb_ref[...] = k_ref[...].astype(jnp.bfloat16)
            vb_ref[...] = v_ref[...].astype(jnp.bfloat16)

            q_ref[...] = jnp.dot(
                xb_ref[...], wq_ref[...], preferred_element_type=jnp.float32
            ).astype(jnp.bfloat16)

        qb = lax.broadcasted_iota(jnp.int32, (SQ, SQ), 0) // BLK
        kb = lax.broadcasted_iota(jnp.int32, (SQ, SQ), 1) // BLK
        mask = (qb == kb) | (kb == 0) | ((qb + kb) % 3 == 0)
        bias = jnp.where(mask, 0.0, -1e9).astype(jnp.float32)

        def attention(b):
            rows = slice(b * SQ, (b + 1) * SQ)
            for h in range(H):
                if KMODE == "noattn":
                    break
                cols = slice(h * DH, (h + 1) * DH)
                s = lax.dot_general(
                    q_ref[rows, cols], kb_ref[rows, cols],
                    (((1,), (1,)), ((), ())),
                    preferred_element_type=jnp.float32,
                )
                e = jnp.exp(s * 0.125 + bias)
                rsum = jnp.sum(e, axis=1, keepdims=True)
                ctx = jnp.dot(e.astype(jnp.bfloat16), vb_ref[rows, cols],
                              preferred_element_type=jnp.float32)
                ctx = ctx * (1.0 / rsum)
                ctx_ref[rows, cols] = ctx.astype(jnp.bfloat16)

        def outproj(base, recv, ssem, rsem):
            descs = []
            for o in range(N_DEV):
                j = (me + o) % N_DEV
                rows = pl.ds(base + j * SC, SC)
                pc = jnp.dot(ctx_ref[rows, :], wo_ref[...],
                             preferred_element_type=jnp.float32
                             ).astype(jnp.bfloat16)
                part_ref[rows, :] = pc
                if o == 0:
                    recv[me, :, :] = pc
                else:
                    d = pltpu.make_async_remote_copy(
                        src_ref=part_ref.at[rows, :],
                        dst_ref=recv.at[me],
                        send_sem=ssem.at[o - 1],
                        recv_sem=rsem.at[o - 1],
                        device_id=(j,), device_id_type=pl.DeviceIdType.MESH)
                    d.start()
                    descs.append(d)
            return descs

        def reduce_chunk(recv, descs, red_ref):
            acc = recv[me].astype(jnp.float32)
            for o in range(1, N_DEV):
                descs[o - 1].wait_recv()
                acc = acc + recv[(me - o) % N_DEV].astype(jnp.float32)
            red_ref[...] = acc.astype(jnp.bfloat16)
            return acc

        def ag_push(base, red_ref, acc, ssem, rsem):
            myrows = pl.ds(base + me * SC, SC)
            agbuf[myrows, :] = red_ref[...]
            o_ref[myrows, :] = acc
            descs = []
            for o in range(1, N_DEV):
                j = (me + o) % N_DEV
                d = pltpu.make_async_remote_copy(
                    src_ref=red_ref,
                    dst_ref=agbuf.at[myrows, :],
                    send_sem=ssem.at[o - 1],
                    recv_sem=rsem.at[o - 1],
                    device_id=(j,), device_id_type=pl.DeviceIdType.MESH)
                d.start()
                descs.append(d)
            return descs

        def ag_collect(base, descs):
            for o in range(1, N_DEV):
                descs[o - 1].wait_recv()
                rows = pl.ds(base + ((me - o) % N_DEV) * SC, SC)
                o_ref[rows, :] = agbuf[rows, :].astype(jnp.float32)

        with _scope("ph_attn0"):
            attention(0)

        if KMODE == "nocomm":
            for b in range(1, B):
                attention(b)
            for o in range(N_DEV):
                j = (me + o) % N_DEV
                for base in (0, SQ):
                    rows = pl.ds(base + j * SC, SC)
                    o_ref[rows, :] = jnp.dot(
                        ctx_ref[rows, :], wo_ref[...],
                        preferred_element_type=jnp.float32)
            return

        with _scope("ph_barrier"):
            pl.semaphore_wait(bar, N_DEV)

        if KMODE == "justbar":
            for b in range(1, B):
                attention(b)
            for o in range(N_DEV):
                j = (me + o) % N_DEV
                for base in (0, SQ):
                    rows = pl.ds(base + j * SC, SC)
                    o_ref[rows, :] = jnp.dot(
                        ctx_ref[rows, :], wo_ref[...],
                        preferred_element_type=jnp.float32)
            return

        with _scope("ph_outprojA"):
            rsA = outproj(0, recvA, rsA_s, rsA_r)
        with _scope("ph_attn1"):
            attention(1)
        with _scope("ph_outprojB"):
            rsB = outproj(SQ, recvB, rsB_s, rsB_r)
        with _scope("ph_reduceA"):
            accA = reduce_chunk(recvA, rsA, redA_ref)

        if KMODE == "noag":
            accB = reduce_chunk(recvB, rsB, redB_ref)
            o_ref[pl.ds(me * SC, SC), :] = accA
            o_ref[pl.ds(SQ + me * SC, SC), :] = accB
            for d in rsA + rsB:
                d.wait_send()
            return

        with _scope("ph_agA"):
            agA = ag_push(0, redA_ref, accA, agA_s, agA_r)
        with _scope("ph_reduceB"):
            accB = reduce_chunk(recvB, rsB, redB_ref)
        with _scope("ph_agB"):
            agB = ag_push(SQ, redB_ref, accB, agB_s, agB_r)

        with _scope("ph_drain_rs"):
            for d in rsA + rsB:
                d.wait_send()
        with _scope("ph_collectA"):
            ag_collect(0, agA)
        with _scope("ph_collectB"):
            ag_collect(SQ, agB)
        with _scope("ph_drain_ag"):
            for d in agA + agB:
                d.wait_send()

    out = pl.pallas_call(
        body,
        out_shape=jax.ShapeDtypeStruct((ROWS, DM), jnp.float32),
        in_specs=[pl.BlockSpec(memory_space=pltpu.VMEM)] * 5,
        out_specs=pl.BlockSpec(memory_space=pltpu.VMEM),
        scratch_shapes=[
            pltpu.VMEM((ROWS, DM), jnp.bfloat16),
            pltpu.VMEM((ROWS, HD), jnp.bfloat16),
            pltpu.VMEM((ROWS, HD), jnp.bfloat16),
            pltpu.VMEM((ROWS, HD), jnp.bfloat16),
            pltpu.VMEM((ROWS, HD), jnp.bfloat16),
            pltpu.VMEM((ROWS, DM), jnp.bfloat16),
            pltpu.VMEM((SC, DM), jnp.bfloat16),
            pltpu.VMEM((SC, DM), jnp.bfloat16),
            pltpu.VMEM((N_DEV, SC, DM), jnp.bfloat16),
            pltpu.VMEM((N_DEV, SC, DM), jnp.bfloat16),
            pltpu.VMEM((ROWS, DM), jnp.bfloat16),
            pltpu.SemaphoreType.DMA((N_DEV - 1,)),
            pltpu.SemaphoreType.DMA((N_DEV - 1,)),
            pltpu.SemaphoreType.DMA((N_DEV - 1,)),
            pltpu.SemaphoreType.DMA((N_DEV - 1,)),
            pltpu.SemaphoreType.DMA((N_DEV - 1,)),
            pltpu.SemaphoreType.DMA((N_DEV - 1,)),
            pltpu.SemaphoreType.DMA((N_DEV - 1,)),
            pltpu.SemaphoreType.DMA((N_DEV - 1,)),
        ],
        compiler_params=pltpu.CompilerParams(
            collective_id=None if KMODE == "nocomm" else 0),
    )(x2, wq_s, k2, v2, wo_s)

    return out.reshape(B, SQ, DM)
